# baseline (speedup 1.0000x reference)
import jax
import jax.numpy as jnp
from jax import lax
from jax.experimental import pallas as pl
from jax.experimental.pallas import tpu as pltpu
from jax.experimental.pallas import tpu_sc as plsc

B, S, C = 128, 32, 8192
R = B * S
B_SC = 32
B_TC = B - B_SC
R_TC = B_TC * S
RB = 256
NEG_INF = float("-inf")
_LN2 = 0.6931471805599453
_SQRT2 = 1.4142135623730951
NW = 32
UN = 16


def _vlog16(x):
    bits = plsc.bitcast(x, jnp.int32)
    e = lax.shift_right_logical(bits, 23) - 127
    mant = lax.bitwise_or(lax.bitwise_and(bits, 0x007FFFFF), 0x3F800000)
    m = plsc.bitcast(mant, jnp.float32)
    big = m > _SQRT2
    m = jnp.where(big, m * 0.5, m)
    ef = e.astype(jnp.float32) + jnp.where(big, 1.0, 0.0)
    t = (m - 1.0) / (m + 1.0)
    t2 = t * t
    p = 2.0 * t * (1.0 + t2 * (1.0 / 3.0 + t2 * (1.0 / 5.0 + t2 * (1.0 / 7.0))))
    return ef * _LN2 + p


def _worker_id():
    return lax.axis_index("s") * 2 + lax.axis_index("c")


def _sc_compute_row(rowbuf, idx_lo, idx_hi, i, lane, ninf16):
    v0 = rowbuf[pl.ds(0, 16)]
    rowbuf[pl.ds(0, 16)] = jnp.where(lane < 2, NEG_INF, v0)
    plsc.store_scatter(rowbuf, [idx_lo], ninf16, mask=lane <= i)
    plsc.store_scatter(rowbuf, [idx_hi], ninf16, mask=(lane + 16) <= i)

    @plsc.parallel_loop(0, C, 16 * UN, carry=(ninf16,) * UN)
    def p1(k, accs):
        return tuple(
            jnp.maximum(accs[t], rowbuf[pl.ds(k + 16 * t, 16)])
            for t in range(UN)
        )

    red = list(p1)
    while len(red) > 1:
        red = [jnp.maximum(red[2 * j], red[2 * j + 1])
               for j in range(len(red) // 2)]
    mv = jnp.full((16,), lax.reduce_max(red[0], axes=(0,)))

    z16 = jnp.zeros((16,), jnp.float32)

    @plsc.parallel_loop(0, C, 16 * UN, carry=(z16,) * UN)
    def p2(k, accs):
        return tuple(
            accs[t] + jnp.exp(rowbuf[pl.ds(k + 16 * t, 16)] - mv)
            for t in range(UN)
        )

    red = list(p2)
    while len(red) > 1:
        red = [red[2 * j] + red[2 * j + 1] for j in range(len(red) // 2)]
    sv = jnp.full((16,), lax.reduce_sum(red[0], axes=(0,)))
    cv = mv + _vlog16(sv)

    @plsc.parallel_loop(0, C, 16 * UN)
    def p3(k):
        for t in range(UN):
            sl = pl.ds(k + 16 * t, 16)
            rowbuf[sl] = rowbuf[sl] - cv


def _sc_body(logits_hbm, idx_hbm, out_hbm, idxrow, buf0, buf1,
             sem_in0, sem_in1, sem_out0, sem_out1):
    wid = _worker_id()
    lane = lax.iota(jnp.int32, 16)
    ninf16 = jnp.full((16,), NEG_INF, jnp.float32)

    b = B_TC + wid
    pltpu.sync_copy(idx_hbm.at[b], idxrow)
    idx_lo = idxrow[pl.ds(0, 16)]
    idx_hi = idxrow[pl.ds(16, 16)]

    bufs = (buf0, buf1)
    sems_in = (sem_in0, sem_in1)
    sems_out = (sem_out0, sem_out1)
    row0 = b * S
    in_flight = [None, None]
    out_flight = [None, None]
    in_flight[0] = pltpu.async_copy(logits_hbm.at[row0], buf0, sem_in0)
    for i in range(S):
        p = i % 2
        q = (i + 1) % 2
        in_flight[p].wait()
        if i + 1 < S:
            if out_flight[q] is not None:
                out_flight[q].wait()
            in_flight[q] = pltpu.async_copy(
                logits_hbm.at[row0 + i + 1], bufs[q], sems_in[q])
        _sc_compute_row(bufs[p], idx_lo, idx_hi, i, lane, ninf16)
        out_flight[p] = pltpu.async_copy(
            bufs[p], out_hbm.at[row0 + i], sems_out[p])
    for p in range(2):
        if out_flight[p] is not None:
            out_flight[p].wait()


def _sc_call(logits2, tgt_in_idx):
    run = pl.kernel(
        _sc_body,
        out_type=jax.ShapeDtypeStruct((R, C), jnp.float32),
        mesh=plsc.VectorSubcoreMesh(core_axis_name="c", subcore_axis_name="s",
                                    num_cores=2, num_subcores=16),
        scratch_types=[
            pltpu.VMEM((S,), jnp.int32),
            pltpu.VMEM((C,), jnp.float32),
            pltpu.VMEM((C,), jnp.float32),
            pltpu.SemaphoreType.DMA,
            pltpu.SemaphoreType.DMA,
            pltpu.SemaphoreType.DMA,
            pltpu.SemaphoreType.DMA,
        ],
        compiler_params=pltpu.CompilerParams(needs_layout_passes=False),
    )
    return run(logits2, tgt_in_idx)


def _tc_body(idx_ref, x_ref, o_ref):
    x = x_ref[...]
    idxcol = idx_ref[...]
    cand = lax.broadcasted_iota(jnp.int32, (RB, C), 1)
    eq = (cand == idxcol).astype(jnp.float32)
    row = lax.broadcasted_iota(jnp.int32, (RB, RB), 0)
    col = lax.broadcasted_iota(jnp.int32, (RB, RB), 1)
    same_b = (row // S) == (col // S)
    tril = (same_b & (row >= col)).astype(jnp.float32)
    counts = jnp.dot(tril, eq, preferred_element_type=jnp.float32)
    mask = (counts > 0.0) | (cand < 2)
    masked = jnp.where(mask, NEG_INF, x)
    m = jnp.max(masked, axis=1, keepdims=True)
    s = jnp.sum(jnp.exp(masked - m), axis=1, keepdims=True)
    o_ref[...] = masked - (m + jnp.log(s))


def _tc_call(x2, idx2, partial_out):
    return pl.pallas_call(
        _tc_body_wrap,
        grid=(R_TC // RB,),
        in_specs=[
            pl.BlockSpec((RB, 1), lambda r: (r, 0)),
            pl.BlockSpec((RB, C), lambda r: (r, 0)),
            pl.BlockSpec(memory_space=pl.ANY),
        ],
        out_specs=pl.BlockSpec((RB, C), lambda r: (r, 0)),
        out_shape=jax.ShapeDtypeStruct((R, C), jnp.float32),
        input_output_aliases={2: 0},
        compiler_params=pltpu.CompilerParams(
            dimension_semantics=("arbitrary",),
        ),
    )(idx2, x2, partial_out)


def _tc_body_wrap(idx_ref, x_ref, alias_ref, o_ref):
    _tc_body(idx_ref, x_ref, o_ref)


def kernel(logits, tgt_in_idx):
    x2 = logits.reshape(R, C)
    idx2 = tgt_in_idx.reshape(R, 1)
    partial = _sc_call(x2, tgt_in_idx)
    return _tc_call(x2, idx2, partial).reshape(B, S, C)


# speedup vs baseline: 1.1531x; 1.1531x over previous
import jax
import jax.numpy as jnp
from jax import lax
from jax.experimental import pallas as pl
from jax.experimental.pallas import tpu as pltpu
from jax.experimental.pallas import tpu_sc as plsc

B, S, C = 128, 32, 8192
R = B * S
B_SC = 16
B_TC = B - B_SC
R_TC = B_TC * S
RB = 256
NEG_INF = float("-inf")
_LN2 = 0.6931471805599453
_SQRT2 = 1.4142135623730951
NW = 32
UN = 8


def _vlog16(x):
    bits = plsc.bitcast(x, jnp.int32)
    e = lax.shift_right_logical(bits, 23) - 127
    mant = lax.bitwise_or(lax.bitwise_and(bits, 0x007FFFFF), 0x3F800000)
    m = plsc.bitcast(mant, jnp.float32)
    big = m > _SQRT2
    m = jnp.where(big, m * 0.5, m)
    ef = e.astype(jnp.float32) + jnp.where(big, 1.0, 0.0)
    t = (m - 1.0) / (m + 1.0)
    t2 = t * t
    p = 2.0 * t * (1.0 + t2 * (1.0 / 3.0 + t2 * (1.0 / 5.0 + t2 * (1.0 / 7.0))))
    return ef * _LN2 + p


def _worker_id():
    return lax.axis_index("s") * 2 + lax.axis_index("c")


def _sc_compute_row(rowbuf, idx_lo, idx_hi, i, lane, ninf16):
    v0 = rowbuf[pl.ds(0, 16)]
    rowbuf[pl.ds(0, 16)] = jnp.where(lane < 2, NEG_INF, v0)
    plsc.store_scatter(rowbuf, [idx_lo], ninf16, mask=lane <= i)
    plsc.store_scatter(rowbuf, [idx_hi], ninf16, mask=(lane + 16) <= i)

    @plsc.parallel_loop(0, C, 16 * UN, carry=(ninf16,) * UN)
    def p1(k, accs):
        return tuple(
            jnp.maximum(accs[t], rowbuf[pl.ds(k + 16 * t, 16)])
            for t in range(UN)
        )

    red = list(p1)
    while len(red) > 1:
        red = [jnp.maximum(red[2 * j], red[2 * j + 1])
               for j in range(len(red) // 2)]
    mv = jnp.full((16,), lax.reduce_max(red[0], axes=(0,)))

    z16 = jnp.zeros((16,), jnp.float32)

    @plsc.parallel_loop(0, C, 16 * UN, carry=(z16,) * UN)
    def p2(k, accs):
        return tuple(
            accs[t] + jnp.exp(rowbuf[pl.ds(k + 16 * t, 16)] - mv)
            for t in range(UN)
        )

    red = list(p2)
    while len(red) > 1:
        red = [red[2 * j] + red[2 * j + 1] for j in range(len(red) // 2)]
    sv = jnp.full((16,), lax.reduce_sum(red[0], axes=(0,)))
    cv = mv + _vlog16(sv)

    @plsc.parallel_loop(0, C, 16 * UN)
    def p3(k):
        for t in range(UN):
            sl = pl.ds(k + 16 * t, 16)
            rowbuf[sl] = rowbuf[sl] - cv


def _sc_body(logits_hbm, idx_hbm, out_hbm, idxrow, buf0, buf1,
             sem_in0, sem_in1, sem_out0, sem_out1):
    wid = _worker_id()
    lane = lax.iota(jnp.int32, 16)
    ninf16 = jnp.full((16,), NEG_INF, jnp.float32)

    HPW = S // 2
    b = B_TC + wid // 2
    i0 = (wid % 2) * HPW
    pltpu.sync_copy(idx_hbm.at[b], idxrow)
    idx_lo = idxrow[pl.ds(0, 16)]
    idx_hi = idxrow[pl.ds(16, 16)]

    bufs = (buf0, buf1)
    sems_in = (sem_in0, sem_in1)
    sems_out = (sem_out0, sem_out1)
    row0 = b * S + i0
    in_flight = [None, None]
    out_flight = [None, None]
    in_flight[0] = pltpu.async_copy(logits_hbm.at[row0], buf0, sem_in0)
    for j in range(HPW):
        p = j % 2
        q = (j + 1) % 2
        in_flight[p].wait()
        if j + 1 < HPW:
            if out_flight[q] is not None:
                out_flight[q].wait()
            in_flight[q] = pltpu.async_copy(
                logits_hbm.at[row0 + j + 1], bufs[q], sems_in[q])
        _sc_compute_row(bufs[p], idx_lo, idx_hi, i0 + j, lane, ninf16)
        out_flight[p] = pltpu.async_copy(
            bufs[p], out_hbm.at[row0 + j], sems_out[p])
    for p in range(2):
        if out_flight[p] is not None:
            out_flight[p].wait()


def _sc_call(logits2, tgt_in_idx):
    run = pl.kernel(
        _sc_body,
        out_type=jax.ShapeDtypeStruct((R, C), jnp.float32),
        mesh=plsc.VectorSubcoreMesh(core_axis_name="c", subcore_axis_name="s",
                                    num_cores=2, num_subcores=16),
        scratch_types=[
            pltpu.VMEM((S,), jnp.int32),
            pltpu.VMEM((C,), jnp.float32),
            pltpu.VMEM((C,), jnp.float32),
            pltpu.SemaphoreType.DMA,
            pltpu.SemaphoreType.DMA,
            pltpu.SemaphoreType.DMA,
            pltpu.SemaphoreType.DMA,
        ],
        compiler_params=pltpu.CompilerParams(needs_layout_passes=False),
    )
    return run(logits2, tgt_in_idx)


def _tc_body(idx_ref, x_ref, o_ref):
    x = x_ref[...]
    idxcol = idx_ref[...]
    cand = lax.broadcasted_iota(jnp.int32, (RB, C), 1)
    eq = (cand == idxcol).astype(jnp.float32)
    row = lax.broadcasted_iota(jnp.int32, (RB, RB), 0)
    col = lax.broadcasted_iota(jnp.int32, (RB, RB), 1)
    same_b = (row // S) == (col // S)
    tril = (same_b & (row >= col)).astype(jnp.float32)
    counts = jnp.dot(tril, eq, preferred_element_type=jnp.float32)
    mask = (counts > 0.0) | (cand < 2)
    masked = jnp.where(mask, NEG_INF, x)
    m = jnp.max(masked, axis=1, keepdims=True)
    s = jnp.sum(jnp.exp(masked - m), axis=1, keepdims=True)
    o_ref[...] = masked - (m + jnp.log(s))


def _tc_call(x2, idx2, partial_out):
    return pl.pallas_call(
        _tc_body_wrap,
        grid=(R_TC // RB,),
        in_specs=[
            pl.BlockSpec((RB, 1), lambda r: (r, 0)),
            pl.BlockSpec((RB, C), lambda r: (r, 0)),
            pl.BlockSpec(memory_space=pl.ANY),
        ],
        out_specs=pl.BlockSpec((RB, C), lambda r: (r, 0)),
        out_shape=jax.ShapeDtypeStruct((R, C), jnp.float32),
        input_output_aliases={2: 0},
        compiler_params=pltpu.CompilerParams(
            dimension_semantics=("arbitrary",),
        ),
    )(idx2, x2, partial_out)


def _tc_body_wrap(idx_ref, x_ref, alias_ref, o_ref):
    _tc_body(idx_ref, x_ref, o_ref)


def kernel(logits, tgt_in_idx):
    x2 = logits.reshape(R, C)
    idx2 = tgt_in_idx.reshape(R, 1)
    partial = _sc_call(x2, tgt_in_idx)
    return _tc_call(x2, idx2, partial).reshape(B, S, C)


# speedup vs baseline: 1.2326x; 1.0689x over previous
import jax
import jax.numpy as jnp
from jax import lax
from jax.experimental import pallas as pl
from jax.experimental.pallas import tpu as pltpu
from jax.experimental.pallas import tpu_sc as plsc

B, S, C = 128, 32, 8192
R = B * S
B_SC = 8
B_TC = B - B_SC
R_TC = B_TC * S
RB = 256
NEG_INF = float("-inf")
_LN2 = 0.6931471805599453
_SQRT2 = 1.4142135623730951
NW = 32
UN = 8


def _vlog16(x):
    bits = plsc.bitcast(x, jnp.int32)
    e = lax.shift_right_logical(bits, 23) - 127
    mant = lax.bitwise_or(lax.bitwise_and(bits, 0x007FFFFF), 0x3F800000)
    m = plsc.bitcast(mant, jnp.float32)
    big = m > _SQRT2
    m = jnp.where(big, m * 0.5, m)
    ef = e.astype(jnp.float32) + jnp.where(big, 1.0, 0.0)
    t = (m - 1.0) / (m + 1.0)
    t2 = t * t
    p = 2.0 * t * (1.0 + t2 * (1.0 / 3.0 + t2 * (1.0 / 5.0 + t2 * (1.0 / 7.0))))
    return ef * _LN2 + p


def _worker_id():
    return lax.axis_index("s") * 2 + lax.axis_index("c")


def _sc_compute_row(rowbuf, idx_lo, idx_hi, i, lane, ninf16):
    v0 = rowbuf[pl.ds(0, 16)]
    rowbuf[pl.ds(0, 16)] = jnp.where(lane < 2, NEG_INF, v0)
    plsc.store_scatter(rowbuf, [idx_lo], ninf16, mask=lane <= i)
    plsc.store_scatter(rowbuf, [idx_hi], ninf16, mask=(lane + 16) <= i)

    @plsc.parallel_loop(0, C, 16 * UN, carry=(ninf16,) * UN)
    def p1(k, accs):
        return tuple(
            jnp.maximum(accs[t], rowbuf[pl.ds(k + 16 * t, 16)])
            for t in range(UN)
        )

    red = list(p1)
    while len(red) > 1:
        red = [jnp.maximum(red[2 * j], red[2 * j + 1])
               for j in range(len(red) // 2)]
    mv = jnp.full((16,), lax.reduce_max(red[0], axes=(0,)))

    z16 = jnp.zeros((16,), jnp.float32)

    @plsc.parallel_loop(0, C, 16 * UN, carry=(z16,) * UN)
    def p2(k, accs):
        return tuple(
            accs[t] + jnp.exp(rowbuf[pl.ds(k + 16 * t, 16)] - mv)
            for t in range(UN)
        )

    red = list(p2)
    while len(red) > 1:
        red = [red[2 * j] + red[2 * j + 1] for j in range(len(red) // 2)]
    sv = jnp.full((16,), lax.reduce_sum(red[0], axes=(0,)))
    cv = mv + _vlog16(sv)

    @plsc.parallel_loop(0, C, 16 * UN)
    def p3(k):
        for t in range(UN):
            sl = pl.ds(k + 16 * t, 16)
            rowbuf[sl] = rowbuf[sl] - cv


def _sc_body(logits_hbm, idx_hbm, out_hbm, idxrow, buf0, buf1,
             sem_in0, sem_in1, sem_out0, sem_out1):
    wid = _worker_id()
    lane = lax.iota(jnp.int32, 16)
    ninf16 = jnp.full((16,), NEG_INF, jnp.float32)

    HPW = B_SC * S // NW
    wpb = NW // B_SC
    b = B_TC + wid // wpb
    i0 = (wid % wpb) * HPW
    pltpu.sync_copy(idx_hbm.at[b], idxrow)
    idx_lo = idxrow[pl.ds(0, 16)]
    idx_hi = idxrow[pl.ds(16, 16)]

    bufs = (buf0, buf1)
    sems_in = (sem_in0, sem_in1)
    sems_out = (sem_out0, sem_out1)
    row0 = b * S + i0
    in_flight = [None, None]
    out_flight = [None, None]
    in_flight[0] = pltpu.async_copy(logits_hbm.at[row0], buf0, sem_in0)
    for j in range(HPW):
        p = j % 2
        q = (j + 1) % 2
        in_flight[p].wait()
        if j + 1 < HPW:
            if out_flight[q] is not None:
                out_flight[q].wait()
            in_flight[q] = pltpu.async_copy(
                logits_hbm.at[row0 + j + 1], bufs[q], sems_in[q])
        _sc_compute_row(bufs[p], idx_lo, idx_hi, i0 + j, lane, ninf16)
        out_flight[p] = pltpu.async_copy(
            bufs[p], out_hbm.at[row0 + j], sems_out[p])
    for p in range(2):
        if out_flight[p] is not None:
            out_flight[p].wait()


def _sc_call(logits2, tgt_in_idx):
    run = pl.kernel(
        _sc_body,
        out_type=jax.ShapeDtypeStruct((R, C), jnp.float32),
        mesh=plsc.VectorSubcoreMesh(core_axis_name="c", subcore_axis_name="s",
                                    num_cores=2, num_subcores=16),
        scratch_types=[
            pltpu.VMEM((S,), jnp.int32),
            pltpu.VMEM((C,), jnp.float32),
            pltpu.VMEM((C,), jnp.float32),
            pltpu.SemaphoreType.DMA,
            pltpu.SemaphoreType.DMA,
            pltpu.SemaphoreType.DMA,
            pltpu.SemaphoreType.DMA,
        ],
        compiler_params=pltpu.CompilerParams(needs_layout_passes=False),
    )
    return run(logits2, tgt_in_idx)


def _tc_body(idx_ref, x_ref, o_ref):
    x = x_ref[...]
    idxcol = idx_ref[...]
    cand = lax.broadcasted_iota(jnp.int32, (RB, C), 1)
    eq = (cand == idxcol).astype(jnp.float32)
    row = lax.broadcasted_iota(jnp.int32, (RB, RB), 0)
    col = lax.broadcasted_iota(jnp.int32, (RB, RB), 1)
    same_b = (row // S) == (col // S)
    tril = (same_b & (row >= col)).astype(jnp.bfloat16)
    counts = jnp.dot(tril, eq.astype(jnp.bfloat16),
                     preferred_element_type=jnp.float32)
    mask = (counts > 0.0) | (cand < 2)
    masked = jnp.where(mask, NEG_INF, x)
    m = jnp.max(masked, axis=1, keepdims=True)
    s = jnp.sum(jnp.exp(masked - m), axis=1, keepdims=True)
    o_ref[...] = masked - (m + jnp.log(s))


def _tc_call(x2, idx2, partial_out):
    return pl.pallas_call(
        _tc_body_wrap,
        grid=(R_TC // RB,),
        in_specs=[
            pl.BlockSpec((RB, 1), lambda r: (r, 0)),
            pl.BlockSpec((RB, C), lambda r: (r, 0)),
            pl.BlockSpec(memory_space=pl.ANY),
        ],
        out_specs=pl.BlockSpec((RB, C), lambda r: (r, 0)),
        out_shape=jax.ShapeDtypeStruct((R, C), jnp.float32),
        input_output_aliases={2: 0},
        compiler_params=pltpu.CompilerParams(
            dimension_semantics=("arbitrary",),
        ),
    )(idx2, x2, partial_out)


def _tc_body_wrap(idx_ref, x_ref, alias_ref, o_ref):
    _tc_body(idx_ref, x_ref, o_ref)


def kernel(logits, tgt_in_idx):
    x2 = logits.reshape(R, C)
    idx2 = tgt_in_idx.reshape(R, 1)
    partial = _sc_call(x2, tgt_in_idx)
    return _tc_call(x2, idx2, partial).reshape(B, S, C)


# speedup vs baseline: 1.2396x; 1.0057x over previous
import jax
import jax.numpy as jnp
from jax import lax
from jax.experimental import pallas as pl
from jax.experimental.pallas import tpu as pltpu
from jax.experimental.pallas import tpu_sc as plsc

B, S, C = 128, 32, 8192
R = B * S
B_SC = 8
B_TC = B - B_SC
R_TC = B_TC * S
RB = 256
NEG_INF = float("-inf")
_LN2 = 0.6931471805599453
_SQRT2 = 1.4142135623730951
NW = 32
UN = 8


def _vlog16(x):
    bits = plsc.bitcast(x, jnp.int32)
    e = lax.shift_right_logical(bits, 23) - 127
    mant = lax.bitwise_or(lax.bitwise_and(bits, 0x007FFFFF), 0x3F800000)
    m = plsc.bitcast(mant, jnp.float32)
    big = m > _SQRT2
    m = jnp.where(big, m * 0.5, m)
    ef = e.astype(jnp.float32) + jnp.where(big, 1.0, 0.0)
    t = (m - 1.0) / (m + 1.0)
    t2 = t * t
    p = 2.0 * t * (1.0 + t2 * (1.0 / 3.0 + t2 * (1.0 / 5.0 + t2 * (1.0 / 7.0))))
    return ef * _LN2 + p


def _worker_id():
    return lax.axis_index("s") * 2 + lax.axis_index("c")


def _sc_compute_row(rowbuf, idx_lo, idx_hi, i, lane, ninf16):
    v0 = rowbuf[pl.ds(0, 16)]
    rowbuf[pl.ds(0, 16)] = jnp.where(lane < 2, NEG_INF, v0)
    plsc.store_scatter(rowbuf, [idx_lo], ninf16, mask=lane <= i)
    plsc.store_scatter(rowbuf, [idx_hi], ninf16, mask=(lane + 16) <= i)

    @plsc.parallel_loop(0, C, 16 * UN, carry=(ninf16,) * UN)
    def p1(k, accs):
        return tuple(
            jnp.maximum(accs[t], rowbuf[pl.ds(k + 16 * t, 16)])
            for t in range(UN)
        )

    red = list(p1)
    while len(red) > 1:
        red = [jnp.maximum(red[2 * j], red[2 * j + 1])
               for j in range(len(red) // 2)]
    mv = jnp.full((16,), lax.reduce_max(red[0], axes=(0,)))

    z16 = jnp.zeros((16,), jnp.float32)

    @plsc.parallel_loop(0, C, 16 * UN, carry=(z16,) * UN)
    def p2(k, accs):
        return tuple(
            accs[t] + jnp.exp(rowbuf[pl.ds(k + 16 * t, 16)] - mv)
            for t in range(UN)
        )

    red = list(p2)
    while len(red) > 1:
        red = [red[2 * j] + red[2 * j + 1] for j in range(len(red) // 2)]
    sv = jnp.full((16,), lax.reduce_sum(red[0], axes=(0,)))
    cv = mv + _vlog16(sv)

    @plsc.parallel_loop(0, C, 16 * UN)
    def p3(k):
        for t in range(UN):
            sl = pl.ds(k + 16 * t, 16)
            rowbuf[sl] = rowbuf[sl] - cv


def _sc_body(logits_hbm, idx_hbm, out_hbm, idxrow, buf0, buf1,
             sem_in0, sem_in1, sem_out0, sem_out1):
    wid = _worker_id()
    lane = lax.iota(jnp.int32, 16)
    ninf16 = jnp.full((16,), NEG_INF, jnp.float32)

    HPW = B_SC * S // NW
    wpb = NW // B_SC
    b = B_TC + wid // wpb
    i0 = (wid % wpb) * HPW
    pltpu.sync_copy(idx_hbm.at[b], idxrow)
    idx_lo = idxrow[pl.ds(0, 16)]
    idx_hi = idxrow[pl.ds(16, 16)]

    bufs = (buf0, buf1)
    sems_in = (sem_in0, sem_in1)
    sems_out = (sem_out0, sem_out1)
    in_flight = [None, None]
    out_flight = [None, None]
    in_flight[0] = pltpu.async_copy(logits_hbm.at[b, i0], buf0, sem_in0)
    for j in range(HPW):
        p = j % 2
        q = (j + 1) % 2
        in_flight[p].wait()
        if j + 1 < HPW:
            if out_flight[q] is not None:
                out_flight[q].wait()
            in_flight[q] = pltpu.async_copy(
                logits_hbm.at[b, i0 + j + 1], bufs[q], sems_in[q])
        _sc_compute_row(bufs[p], idx_lo, idx_hi, i0 + j, lane, ninf16)
        out_flight[p] = pltpu.async_copy(
            bufs[p], out_hbm.at[b, i0 + j], sems_out[p])
    for p in range(2):
        if out_flight[p] is not None:
            out_flight[p].wait()


def _sc_call(logits, tgt_in_idx):
    run = pl.kernel(
        _sc_body,
        out_type=jax.ShapeDtypeStruct((B, S, C), jnp.float32),
        mesh=plsc.VectorSubcoreMesh(core_axis_name="c", subcore_axis_name="s",
                                    num_cores=2, num_subcores=16),
        scratch_types=[
            pltpu.VMEM((S,), jnp.int32),
            pltpu.VMEM((C,), jnp.float32),
            pltpu.VMEM((C,), jnp.float32),
            pltpu.SemaphoreType.DMA,
            pltpu.SemaphoreType.DMA,
            pltpu.SemaphoreType.DMA,
            pltpu.SemaphoreType.DMA,
        ],
        compiler_params=pltpu.CompilerParams(needs_layout_passes=False),
    )
    return run(logits, tgt_in_idx)


def _tc_math(x, idxcol):
    cand = lax.broadcasted_iota(jnp.int32, (RB, C), 1)
    eq = (cand == idxcol).astype(jnp.float32)
    row = lax.broadcasted_iota(jnp.int32, (RB, RB), 0)
    col = lax.broadcasted_iota(jnp.int32, (RB, RB), 1)
    same_b = (row // S) == (col // S)
    tril = (same_b & (row >= col)).astype(jnp.bfloat16)
    counts = jnp.dot(tril, eq.astype(jnp.bfloat16),
                     preferred_element_type=jnp.float32)
    mask = (counts > 0.0) | (cand < 2)
    masked = jnp.where(mask, NEG_INF, x)
    m = jnp.max(masked, axis=1, keepdims=True)
    s = jnp.sum(jnp.exp(masked - m), axis=1, keepdims=True)
    return masked - (m + jnp.log(s))


BB = RB // S


def _tc_call(logits, idx3, partial_out):
    return pl.pallas_call(
        _tc_body_wrap,
        grid=(B_TC // BB,),
        in_specs=[
            pl.BlockSpec((BB, S, 1), lambda r: (r, 0, 0)),
            pl.BlockSpec((BB, S, C), lambda r: (r, 0, 0)),
            pl.BlockSpec(memory_space=pl.ANY),
        ],
        out_specs=pl.BlockSpec((BB, S, C), lambda r: (r, 0, 0)),
        out_shape=jax.ShapeDtypeStruct((B, S, C), jnp.float32),
        input_output_aliases={2: 0},
        compiler_params=pltpu.CompilerParams(
            dimension_semantics=("arbitrary",),
        ),
    )(idx3, logits, partial_out)


def _tc_body_wrap(idx_ref, x_ref, alias_ref, o_ref):
    x = x_ref[...].reshape(RB, C)
    idxcol = idx_ref[...].reshape(RB, 1)
    o_ref[...] = _tc_math(x, idxcol).reshape(BB, S, C)


def kernel(logits, tgt_in_idx):
    idx3 = tgt_in_idx[:, :, None]
    partial = _sc_call(logits, tgt_in_idx)
    return _tc_call(logits, idx3, partial)
